# baseline (speedup 1.0000x reference)
import functools

import jax
import jax.numpy as jnp
from jax import lax
from jax.experimental import pallas as pl
from jax.experimental.pallas import tpu as pltpu
from jax.experimental.pallas import tpu_sc as plsc

NV = 2048
NB = 2
KNN = 16
_PREC = lax.Precision.DEFAULT
_ROWS = 256


def _ft_body(x_ref, m_ref, w_ref, b_ref, o_ref):
    xm = x_ref[...] * m_ref[0]
    o_ref[0] = jnp.dot(xm, w_ref[0], precision=_PREC) + b_ref[0]


def _ft(x, ft_mask, W_ft, b_ft):
    return pl.pallas_call(
        _ft_body,
        grid=(NB,),
        in_specs=[
            pl.BlockSpec((NV, 16), lambda b: (0, 0)),
            pl.BlockSpec((1, 1, 16), lambda b: (b, 0, 0)),
            pl.BlockSpec((1, 16, 256), lambda b: (b, 0, 0)),
            pl.BlockSpec((1, 1, 256), lambda b: (b, 0, 0)),
        ],
        out_specs=pl.BlockSpec((1, NV, 256), lambda b: (b, 0, 0)),
        out_shape=jax.ShapeDtypeStruct((NB, NV, 256), jnp.float32),
    )(x, ft_mask.reshape(NB, 1, 16), W_ft, b_ft.reshape(NB, 1, 256))


def _pre_body(hfull_ref, hrow_ref, wl_ref, wr_ref, gl_ref, gr_ref, idx_ref):
    b = pl.program_id(0)
    rows = hrow_ref[0]
    hf = hfull_ref[0]
    gl_ref[0] = jnp.dot(rows, wl_ref[...], precision=_PREC)
    gr_ref[0] = jnp.dot(rows, wr_ref[...], precision=_PREC)
    sqf = jnp.sum(hf * hf, axis=-1)
    sqr = jnp.sum(rows * rows, axis=-1)
    mm = lax.dot_general(rows, hf, (((1,), (1,)), ((), ())), precision=_PREC)
    d = (sqr[:, None] + sqf[None, :]) - 2.0 * mm
    cols = lax.broadcasted_iota(jnp.int32, d.shape, 1)
    picks = []
    for _ in range(KNN):
        mn = jnp.min(d, axis=1, keepdims=True)
        ij = jnp.min(jnp.where(d == mn, cols, NV), axis=1, keepdims=True)
        picks.append(ij)
        d = jnp.where(cols == ij, jnp.float32(jnp.inf), d)
    pad = [jnp.zeros((_ROWS, 1), jnp.int32)] * (128 - KNN)
    idx_ref[0] = jnp.concatenate(picks + pad, axis=1) + b * NV


def _pre(h, Wl, Wr):
    din, dout = Wl.shape
    nblk = NV // _ROWS
    return pl.pallas_call(
        _pre_body,
        grid=(NB, nblk),
        in_specs=[
            pl.BlockSpec((1, NV, din), lambda b, r: (b, 0, 0)),
            pl.BlockSpec((1, _ROWS, din), lambda b, r: (b, r, 0)),
            pl.BlockSpec((din, dout), lambda b, r: (0, 0)),
            pl.BlockSpec((din, dout), lambda b, r: (0, 0)),
        ],
        out_specs=[
            pl.BlockSpec((1, _ROWS, dout), lambda b, r: (b, r, 0)),
            pl.BlockSpec((1, _ROWS, dout), lambda b, r: (b, r, 0)),
            pl.BlockSpec((1, _ROWS, 128), lambda b, r: (b, r, 0)),
        ],
        out_shape=[
            jax.ShapeDtypeStruct((NB, NV, dout), jnp.float32),
            jax.ShapeDtypeStruct((NB, NV, dout), jnp.float32),
            jax.ShapeDtypeStruct((NB, NV, 128), jnp.int32),
        ],
    )(h, h, Wl, Wr)


_NCORES = 2
_NSUB = 16
_NW = _NCORES * _NSUB
_GRP = 8
_NPW = (NB * NV) // _NW


def _bfr(x):
    u = plsc.bitcast(x, jnp.uint32)
    r = (u + jnp.uint32(0x7FFF) + ((u >> 16) & jnp.uint32(1))) & jnp.uint32(0xFFFF0000)
    return plsc.bitcast(r, jnp.float32)


def _sc_att_build(dout, interpret=False):
    cchunk = dout // 16
    ngrp = _NPW // _GRP
    mesh = plsc.VectorSubcoreMesh(core_axis_name="c", subcore_axis_name="s")

    def body(gl_hbm, gr_hbm, idx_hbm, a_hbm, out_hbm,
             idx_v, a_v, gl_v, rows_v, out_v, sem):
        wid = lax.axis_index("s") * _NCORES + lax.axis_index("c")
        base = wid * _NPW
        pltpu.sync_copy(idx_hbm.at[pl.ds(base * KNN, _NPW * KNN)], idx_v)
        pltpu.sync_copy(a_hbm, a_v)

        def group(g, _):
            gb = base + g * _GRP
            pltpu.sync_copy(gl_hbm.at[pl.ds(gb, _GRP)], gl_v)
            pltpu.async_copy(
                gr_hbm.at[idx_v.at[pl.ds(g * (_GRP * KNN), _GRP * KNN)]],
                rows_v, sem).wait()
            for n in range(_GRP):
                rbase = n * KNN

                def estep(c, carry, n=n, rbase=rbase):
                    accs, comps = carry
                    co = c * 16
                    glc = gl_v[n, pl.ds(co, 16)]
                    ac = a_v[pl.ds(co, 16)]
                    na, nc = [], []
                    for j in range(KNN):
                        z = glc + rows_v[rbase + j, pl.ds(co, 16)]
                        term = _bfr(jnp.maximum(z, 0.2 * z)) * ac
                        y = term - comps[j]
                        t = accs[j] + y
                        nc.append((t - accs[j]) - y)
                        na.append(t)
                    return tuple(na), tuple(nc)

                zeros = tuple(jnp.zeros((16,), jnp.float32) for _ in range(KNN))
                accs, _ = lax.fori_loop(0, cchunk, estep, (zeros, zeros))
                lane = lax.iota(jnp.int32, 16)
                ev = jnp.zeros((16,), jnp.float32)
                for j in range(KNN):
                    v = accs[j]
                    s = [v[i] for i in range(16)]
                    while len(s) > 1:
                        s = [s[i] + s[i + 1] for i in range(0, len(s), 2)]
                    ev = jnp.where(lane == j, s[0], ev)
                ex = jnp.exp(ev - jnp.max(ev))
                alpha = ex / jnp.sum(ex)
                alphas = [alpha[j] for j in range(KNN)]

                def ostep(c, _, n=n, rbase=rbase, alphas=alphas):
                    co = c * 16
                    acc = alphas[0] * rows_v[rbase, pl.ds(co, 16)]
                    for j in range(1, KNN):
                        acc = acc + alphas[j] * rows_v[rbase + j, pl.ds(co, 16)]
                    out_v[n, pl.ds(co, 16)] = acc
                    return 0

                lax.fori_loop(0, cchunk, ostep, 0)
            pltpu.sync_copy(out_v, out_hbm.at[pl.ds(gb, _GRP)])
            return 0

        lax.fori_loop(0, ngrp, group, 0)

    return pl.kernel(
        body,
        out_type=jax.ShapeDtypeStruct((NB * NV + 8, dout), jnp.float32),
        mesh=mesh,
        interpret=interpret,
        compiler_params=pltpu.CompilerParams(needs_layout_passes=False,
                                             has_side_effects=True),
        scratch_types=[
            pltpu.VMEM((_NPW * KNN,), jnp.int32),
            pltpu.VMEM((dout,), jnp.float32),
            pltpu.VMEM((_GRP, dout), jnp.float32),
            pltpu.VMEM((_GRP * KNN, dout), jnp.float32),
            pltpu.VMEM((_GRP, dout), jnp.float32),
            pltpu.SemaphoreType.DMA,
        ],
    )


def _att_jax(gl, gr, idxg, a):
    nbr = gr[idxg]
    z = jax.nn.leaky_relu(gl[:, None, :] + nbr, negative_slope=0.2)
    e = jnp.einsum('nke,e->nk', z, a)
    alpha = jax.nn.softmax(e, axis=-1)
    return jnp.sum(alpha[..., None] * nbr, axis=1)


def _gat_layer(h, Wl, Wr, a, use_sc):
    dout = Wl.shape[1]
    gl, gr, idx = _pre(h, Wl, Wr)
    glf = gl.reshape(NB * NV, dout)
    grf = gr.reshape(NB * NV, dout)
    if use_sc:
        a_rounded = a.astype(jnp.bfloat16).astype(jnp.float32)
        idxf = idx[:, :, :KNN].reshape(NB * NV * KNN)
        o = _sc_att_build(dout)(glf, grf, idxf, a_rounded)
        o = o[:NB * NV]
    else:
        o = _att_jax(glf, grf, idx[:, :, :KNN].reshape(NB * NV, KNN), a)
    return o.reshape(NB, NV, dout)


def _mlp_body(m_ref, w1_ref, b1_ref, w2_ref, b2_ref, wg_ref, bg_ref, gv_ref, o_ref):
    h = jnp.maximum(jnp.dot(m_ref[...], w1_ref[...], precision=_PREC) + b1_ref[...], 0.0)
    h = jnp.maximum(jnp.dot(h, w2_ref[...], precision=_PREC) + b2_ref[...], 0.0)
    y = jnp.tanh(jnp.dot(h, wg_ref[...], precision=_PREC) + bg_ref[...])
    o_ref[...] = jnp.concatenate(
        [y * gv_ref[...], jnp.zeros((NV, 125), jnp.float32)], axis=1)


def _mlp(merged, W1, b1, W2, b2, Wg_s, bg_s, geod_v):
    out = pl.pallas_call(
        _mlp_body,
        out_shape=jax.ShapeDtypeStruct((NV, 128), jnp.float32),
    )(merged, W1, b1.reshape(1, 256), W2, b2.reshape(1, 64),
      Wg_s, bg_s.reshape(1, 3), geod_v.reshape(NV, 1))
    return out[:, :3]


def kernel(x, ft_mask, W_ft, b_ft, Wl1, Wr1, a1, Wl2, Wr2, a2, Wl3, Wr3, a3,
           Wl4, Wr4, a4, W1, b1, W2, b2, Wg, bg, geod_v, geod_scale):
    h0 = _ft(x, ft_mask, W_ft, b_ft)
    o1 = _gat_layer(h0, Wl1, Wr1, a1, use_sc=False)
    o2 = _gat_layer(jnp.concatenate([h0, o1], axis=-1), Wl2, Wr2, a2, use_sc=False)
    o3 = _gat_layer(jnp.concatenate([o1, o2], axis=-1), Wl3, Wr3, a3, use_sc=False)
    o4 = _gat_layer(jnp.concatenate([o2, o3], axis=-1), Wl4, Wr4, a4, use_sc=True)
    merged = jnp.concatenate([o4[0], o4[1]], axis=-1)
    return _mlp(merged, W1, b1, W2, b2, Wg * geod_scale, bg * geod_scale, geod_v)
